# Initial kernel scaffold; baseline (speedup 1.0000x reference)
import jax
import jax.numpy as jnp
from jax.experimental import pallas as pl


def kernel(sales, item_id, text, global_token, W_sales, b_sales, emb_item, emb_text):
    raise NotImplementedError("write your pallas kernel here")


# speedup vs baseline: 1.2452x; 1.2452x over previous
import functools

import jax
import jax.numpy as jnp
from jax import lax
from jax.experimental import pallas as pl
from jax.experimental.pallas import tpu as pltpu
from jax.experimental.pallas import tpu_sc as plsc

D = 64
C = 128

_BCAST_DNUMS = lax.GatherDimensionNumbers(
    offset_dims=(), collapsed_slice_dims=(0,), start_index_map=(0,))


def _lane_broadcast(vec, i):
    idx = jnp.full((16, 1), i, dtype=jnp.int32)
    return lax.gather(vec, idx, _BCAST_DNUMS, (1,),
                      mode=lax.GatherScatterMode.PROMISE_IN_BOUNDS)


@functools.partial(jax.jit, static_argnames=("n_tokens",))
def _sc_embed(sales_f, item2, text2, consts, tab_i, tab_t, *, n_tokens):
    info = plsc.get_sparse_core_info()
    nc, ns = info.num_cores, info.num_subcores
    nw = nc * ns
    tpw = n_tokens // nw
    nch = tpw // C

    mesh = plsc.VectorSubcoreMesh(core_axis_name="c", subcore_axis_name="s")

    @functools.partial(
        pl.kernel,
        mesh=mesh,
        out_type=jax.ShapeDtypeStruct((n_tokens, 4 * D), jnp.float32),
        scratch_types=[
            pltpu.VMEM((1, C), jnp.int32),
            pltpu.VMEM((1, C), jnp.int32),
            pltpu.VMEM((1, C), jnp.int32),
            pltpu.VMEM((1, C), jnp.int32),
            pltpu.VMEM((C,), jnp.float32),
            pltpu.VMEM((C, 128), jnp.float32),
            pltpu.VMEM((C, 128), jnp.float32),
            pltpu.VMEM((C, 4 * D), jnp.float32),
            pltpu.VMEM((3 * D,), jnp.float32),
            pltpu.SemaphoreType.DMA,
        ],
    )
    def body(sales_hbm, item_hbm, text_hbm, consts_hbm, tab_i_hbm, tab_t_hbm,
             out_hbm, idx_i, idx_t, slab_i, slab_t, sal_v, slabs_i, slabs_t,
             block, cst, sem):
        wid = lax.axis_index("s") * nc + lax.axis_index("c")
        base0 = wid * tpw
        r0_0 = wid * nch

        pltpu.sync_copy(consts_hbm, cst)
        g = [cst[pl.ds(k * 16, 16)] for k in range(D // 16)]
        w = [cst[pl.ds(D + k * 16, 16)] for k in range(D // 16)]
        b = [cst[pl.ds(2 * D + k * 16, 16)] for k in range(D // 16)]

        def fill_g(t, carry):
            for k in range(D // 16):
                block[t, pl.ds(k * 16, 16)] = g[k]
            return carry

        lax.fori_loop(0, C, fill_g, 0)

        def chunk(ci, carry):
            base = base0 + ci * C
            r0 = r0_0 + ci
            pltpu.sync_copy(item_hbm.at[pl.ds(r0, 1)], idx_i)
            pltpu.sync_copy(text_hbm.at[pl.ds(r0, 1)], idx_t)
            pltpu.sync_copy(sales_hbm.at[pl.ds(base, C)], sal_v)

            def mkslab(i, carry):
                slab_i[0, pl.ds(i * 16, 16)] = idx_i[0, pl.ds(i * 16, 16)] >> 1
                slab_t[0, pl.ds(i * 16, 16)] = idx_t[0, pl.ds(i * 16, 16)] >> 1
                return carry

            lax.fori_loop(0, C // 16, mkslab, 0)
            cp_i = pltpu.async_copy(tab_i_hbm.at[slab_i.at[0]], slabs_i, sem)
            cp_t = pltpu.async_copy(tab_t_hbm.at[slab_t.at[0]], slabs_t, sem)

            def grp(gi, carry):
                t0 = gi * 16
                sv16 = sal_v[pl.ds(t0, 16)]
                for i in range(16):
                    sv = _lane_broadcast(sv16, i)
                    for k in range(D // 16):
                        block[t0 + i, pl.ds(D + k * 16, 16)] = sv * w[k] + b[k]
                return carry

            lax.fori_loop(0, C // 16, grp, 0)

            cp_i.wait()
            cp_t.wait()

            def sel(gi, carry):
                t0 = gi * 16
                hi16 = (idx_i[0, pl.ds(t0, 16)] & 1).astype(jnp.float32)
                ht16 = (idx_t[0, pl.ds(t0, 16)] & 1).astype(jnp.float32)
                for i in range(16):
                    t = t0 + i
                    hfi = _lane_broadcast(hi16, i)
                    hft = _lane_broadcast(ht16, i)
                    for k in range(D // 16):
                        lo = slabs_i[t, pl.ds(k * 16, 16)]
                        hi = slabs_i[t, pl.ds(D + k * 16, 16)]
                        block[t, pl.ds(2 * D + k * 16, 16)] = \
                            lo + hfi * (hi - lo)
                        lo = slabs_t[t, pl.ds(k * 16, 16)]
                        hi = slabs_t[t, pl.ds(D + k * 16, 16)]
                        block[t, pl.ds(3 * D + k * 16, 16)] = \
                            lo + hft * (hi - lo)
                return carry

            lax.fori_loop(0, C // 16, sel, 0)

            pltpu.sync_copy(block, out_hbm.at[pl.ds(base, C)])
            return carry

        lax.fori_loop(0, nch, chunk, 0)

    return body(sales_f, item2, text2, consts, tab_i, tab_t)


def kernel(sales, item_id, text, global_token, W_sales, b_sales,
           emb_item, emb_text):
    bsz, seq = item_id.shape
    n = bsz * seq
    sales_f = sales.reshape(bsz, seq).T.reshape(n).astype(jnp.float32)
    item2 = item_id.T.reshape(n // C, C).astype(jnp.int32)
    text2 = text.T.reshape(n // C, C).astype(jnp.int32)
    consts = jnp.concatenate([
        global_token.reshape(-1).astype(jnp.float32),
        W_sales.reshape(-1).astype(jnp.float32),
        b_sales.reshape(-1).astype(jnp.float32),
    ])
    vi = emb_item.shape[0]
    vt = emb_text.shape[0]
    tab_i = (emb_item.T.reshape(D, vi // 2, 2).transpose(1, 2, 0)
             .reshape(vi // 2, 2 * D))
    tab_t = (emb_text.T.reshape(D, vt // 2, 2).transpose(1, 2, 0)
             .reshape(vt // 2, 2 * D))
    out = _sc_embed(sales_f, item2, text2, consts, tab_i, tab_t, n_tokens=n)
    return out.reshape(seq, bsz, 4 * D).transpose(1, 0, 2)


# speedup vs baseline: 1.4542x; 1.1678x over previous
import functools

import jax
import jax.numpy as jnp
from jax import lax
from jax.experimental import pallas as pl
from jax.experimental.pallas import tpu as pltpu
from jax.experimental.pallas import tpu_sc as plsc

D = 64
C = 64

_BCAST_DNUMS = lax.GatherDimensionNumbers(
    offset_dims=(), collapsed_slice_dims=(0,), start_index_map=(0,))


def _lane_broadcast(vec, i):
    idx = jnp.full((16, 1), i, dtype=jnp.int32)
    return lax.gather(vec, idx, _BCAST_DNUMS, (1,),
                      mode=lax.GatherScatterMode.PROMISE_IN_BOUNDS)


@functools.partial(jax.jit, static_argnames=("n_tokens",))
def _sc_embed(sales_f, item2, text2, consts, tab_i, tab_t, *, n_tokens):
    info = plsc.get_sparse_core_info()
    nc, ns = info.num_cores, info.num_subcores
    nw = nc * ns
    tpw = n_tokens // nw
    nch = tpw // C

    mesh = plsc.VectorSubcoreMesh(core_axis_name="c", subcore_axis_name="s")

    @functools.partial(
        pl.kernel,
        mesh=mesh,
        out_type=jax.ShapeDtypeStruct((n_tokens, 4 * D), jnp.float32),
        scratch_types=[
            pltpu.VMEM((nch, C), jnp.int32),
            pltpu.VMEM((nch, C), jnp.int32),
            pltpu.VMEM((2, C), jnp.int32),
            pltpu.VMEM((2, C), jnp.int32),
            pltpu.VMEM((nch, C), jnp.float32),
            pltpu.VMEM((2, C, 128), jnp.float32),
            pltpu.VMEM((2, C, 128), jnp.float32),
            pltpu.VMEM((2, C, 4 * D), jnp.float32),
            pltpu.VMEM((3 * D,), jnp.float32),
            pltpu.SemaphoreType.DMA,
            pltpu.SemaphoreType.DMA,
            pltpu.SemaphoreType.DMA,
            pltpu.SemaphoreType.DMA,
        ],
    )
    def body(sales_hbm, item_hbm, text_hbm, consts_hbm, tab_i_hbm, tab_t_hbm,
             out_hbm, idx_i, idx_t, slab_i, slab_t, sal, slabs_i, slabs_t,
             blocks, cst, gsem0, gsem1, wsem0, wsem1):
        wid = lax.axis_index("s") * nc + lax.axis_index("c")
        base0 = wid * tpw
        gsems = (gsem0, gsem1)
        wsems = (wsem0, wsem1)

        pltpu.sync_copy(consts_hbm, cst)
        pltpu.sync_copy(item_hbm.at[wid], idx_i)
        pltpu.sync_copy(text_hbm.at[wid], idx_t)
        pltpu.sync_copy(sales_hbm.at[wid], sal)

        g = [cst[pl.ds(k * 16, 16)] for k in range(D // 16)]
        w = [cst[pl.ds(D + k * 16, 16)] for k in range(D // 16)]
        b = [cst[pl.ds(2 * D + k * 16, 16)] for k in range(D // 16)]

        def mkslab(ch, p):
            for k in range(C // 16):
                s = pl.ds(k * 16, 16)
                slab_i[p, s] = idx_i[ch, s] >> 1
                slab_t[p, s] = idx_t[ch, s] >> 1

        def fill_g(t, carry):
            for p in range(2):
                for k in range(D // 16):
                    blocks[p, t, pl.ds(k * 16, 16)] = g[k]
            return carry

        lax.fori_loop(0, C, fill_g, 0)

        def fire(ch, p):
            mkslab(ch, p)
            pltpu.async_copy(tab_i_hbm.at[slab_i.at[p]],
                             slabs_i.at[p], gsems[p])
            pltpu.async_copy(tab_t_hbm.at[slab_t.at[p]],
                             slabs_t.at[p], gsems[p])

        def gwait(p):
            pltpu.make_async_copy(tab_i_hbm.at[slab_i.at[p]],
                                  slabs_i.at[p], gsems[p]).wait()
            pltpu.make_async_copy(tab_t_hbm.at[slab_t.at[p]],
                                  slabs_t.at[p], gsems[p]).wait()

        fire(0, 0)
        fire(1, 1)

        def duo(gg, carry):
            for p in range(2):
                ch = gg * 2 + p
                base = base0 + ch * C
                gwait(p)

                @pl.when(ch >= 2)
                def _():
                    pltpu.make_async_copy(
                        blocks.at[p], out_hbm.at[pl.ds(base - 2 * C, C)],
                        wsems[p]).wait()

                def grp(gi, carry2):
                    t0 = gi * 16
                    sv16 = sal[ch, pl.ds(t0, 16)]
                    hi16 = (idx_i[ch, pl.ds(t0, 16)] & 1).astype(jnp.float32)
                    ht16 = (idx_t[ch, pl.ds(t0, 16)] & 1).astype(jnp.float32)
                    for i in range(16):
                        t = t0 + i
                        sv = _lane_broadcast(sv16, i)
                        hfi = _lane_broadcast(hi16, i)
                        hft = _lane_broadcast(ht16, i)
                        for k in range(D // 16):
                            blocks[p, t, pl.ds(D + k * 16, 16)] = \
                                sv * w[k] + b[k]
                            lo = slabs_i[p, t, pl.ds(k * 16, 16)]
                            hi = slabs_i[p, t, pl.ds(D + k * 16, 16)]
                            blocks[p, t, pl.ds(2 * D + k * 16, 16)] = \
                                lo + hfi * (hi - lo)
                            lo = slabs_t[p, t, pl.ds(k * 16, 16)]
                            hi = slabs_t[p, t, pl.ds(D + k * 16, 16)]
                            blocks[p, t, pl.ds(3 * D + k * 16, 16)] = \
                                lo + hft * (hi - lo)
                    return carry2

                lax.fori_loop(0, C // 16, grp, 0)

                @pl.when(ch + 2 < nch)
                def _():
                    fire(ch + 2, p)

                pltpu.async_copy(blocks.at[p], out_hbm.at[pl.ds(base, C)],
                                 wsems[p])
            return carry

        lax.fori_loop(0, nch // 2, duo, 0)

        for p in range(2):
            ch = nch - 2 + p
            pltpu.make_async_copy(
                blocks.at[p], out_hbm.at[pl.ds(base0 + ch * C, C)],
                wsems[p]).wait()

    return body(sales_f, item2, text2, consts, tab_i, tab_t)


def kernel(sales, item_id, text, global_token, W_sales, b_sales,
           emb_item, emb_text):
    bsz, seq = item_id.shape
    n = bsz * seq
    nw = 32
    sales_f = (sales.reshape(bsz, seq).T
               .reshape(nw, n // (nw * C), C).astype(jnp.float32))
    item2 = item_id.T.reshape(nw, n // (nw * C), C).astype(jnp.int32)
    text2 = text.T.reshape(nw, n // (nw * C), C).astype(jnp.int32)
    consts = jnp.concatenate([
        global_token.reshape(-1).astype(jnp.float32),
        W_sales.reshape(-1).astype(jnp.float32),
        b_sales.reshape(-1).astype(jnp.float32),
    ])
    vi = emb_item.shape[0]
    vt = emb_text.shape[0]
    tab_i = (emb_item.T.reshape(D, vi // 2, 2).transpose(1, 2, 0)
             .reshape(vi // 2, 2 * D))
    tab_t = (emb_text.T.reshape(D, vt // 2, 2).transpose(1, 2, 0)
             .reshape(vt // 2, 2 * D))
    out = _sc_embed(sales_f, item2, text2, consts, tab_i, tab_t, n_tokens=n)
    return out.reshape(seq, bsz, 4 * D).transpose(1, 0, 2)


# speedup vs baseline: 2.5445x; 1.7497x over previous
import functools

import jax
import jax.numpy as jnp
from jax import lax
from jax.experimental import pallas as pl
from jax.experimental.pallas import tpu as pltpu
from jax.experimental.pallas import tpu_sc as plsc

D = 64
C = 64
TB = 2048

_BCAST_DNUMS = lax.GatherDimensionNumbers(
    offset_dims=(), collapsed_slice_dims=(0,), start_index_map=(0,))


def _lane_broadcast(vec, i):
    idx = jnp.full((16, 1), i, dtype=jnp.int32)
    return lax.gather(vec, idx, _BCAST_DNUMS, (1,),
                      mode=lax.GatherScatterMode.PROMISE_IN_BOUNDS)


def _repack_body(t_ref, out_ref):
    out_ref[:, 0:D] = t_ref[...].T


def _tc_repack(table):
    v, d = table.shape
    grid = (v + TB - 1) // TB
    return pl.pallas_call(
        _repack_body,
        grid=(grid,),
        in_specs=[pl.BlockSpec((d, TB), lambda g: (0, g))],
        out_specs=pl.BlockSpec((TB, 2 * d), lambda g: (g, 0)),
        out_shape=jax.ShapeDtypeStruct((v, 2 * d), jnp.float32),
    )(table.T)


@functools.partial(jax.jit, static_argnames=("n_tokens",))
def _sc_embed(sales_f, item2, text2, consts, tab_i, tab_t, *, n_tokens):
    info = plsc.get_sparse_core_info()
    nc, ns = info.num_cores, info.num_subcores
    nw = nc * ns
    tpw = n_tokens // nw
    nch = tpw // C

    mesh = plsc.VectorSubcoreMesh(core_axis_name="c", subcore_axis_name="s")

    @functools.partial(
        pl.kernel,
        mesh=mesh,
        out_type=jax.ShapeDtypeStruct((n_tokens, 4 * D), jnp.float32),
        scratch_types=[
            pltpu.VMEM((nch, C), jnp.int32),
            pltpu.VMEM((nch, C), jnp.int32),
            pltpu.VMEM((nch, C), jnp.float32),
            pltpu.VMEM((2, C, 128), jnp.float32),
            pltpu.VMEM((2, C, 128), jnp.float32),
            pltpu.VMEM((2, C, 4 * D), jnp.float32),
            pltpu.VMEM((3 * D,), jnp.float32),
            pltpu.SemaphoreType.DMA,
            pltpu.SemaphoreType.DMA,
            pltpu.SemaphoreType.DMA,
            pltpu.SemaphoreType.DMA,
        ],
    )
    def body(sales_hbm, item_hbm, text_hbm, consts_hbm, tab_i_hbm, tab_t_hbm,
             out_hbm, idx_i, idx_t, sal, rows_i, rows_t,
             blocks, cst, gsem0, gsem1, wsem0, wsem1):
        wid = lax.axis_index("s") * nc + lax.axis_index("c")
        base0 = wid * tpw
        gsems = (gsem0, gsem1)
        wsems = (wsem0, wsem1)

        pltpu.sync_copy(consts_hbm, cst)
        pltpu.sync_copy(item_hbm.at[wid], idx_i)
        pltpu.sync_copy(text_hbm.at[wid], idx_t)
        pltpu.sync_copy(sales_hbm.at[wid], sal)

        g = [cst[pl.ds(k * 16, 16)] for k in range(D // 16)]
        w = [cst[pl.ds(D + k * 16, 16)] for k in range(D // 16)]
        b = [cst[pl.ds(2 * D + k * 16, 16)] for k in range(D // 16)]

        def fill_g(t, carry):
            for p in range(2):
                for k in range(D // 16):
                    blocks[p, t, pl.ds(k * 16, 16)] = g[k]
            return carry

        lax.fori_loop(0, C, fill_g, 0)

        def fire(ch, p):
            pltpu.async_copy(tab_i_hbm.at[idx_i.at[ch]],
                             rows_i.at[p], gsems[p])
            pltpu.async_copy(tab_t_hbm.at[idx_t.at[ch]],
                             rows_t.at[p], gsems[p])

        def gwait(ch, p):
            pltpu.make_async_copy(tab_i_hbm.at[idx_i.at[ch]],
                                  rows_i.at[p], gsems[p]).wait()
            pltpu.make_async_copy(tab_t_hbm.at[idx_t.at[ch]],
                                  rows_t.at[p], gsems[p]).wait()

        fire(0, 0)
        fire(1, 1)

        def duo(gg, carry):
            for p in range(2):
                ch = gg * 2 + p
                base = base0 + ch * C
                gwait(ch, p)

                @pl.when(ch >= 2)
                def _():
                    pltpu.make_async_copy(
                        blocks.at[p], out_hbm.at[pl.ds(base - 2 * C, C)],
                        wsems[p]).wait()

                def grp(gi, carry2):
                    t0 = gi * 16
                    sv16 = sal[ch, pl.ds(t0, 16)]
                    for i in range(16):
                        t = t0 + i
                        sv = _lane_broadcast(sv16, i)
                        for k in range(D // 16):
                            s = pl.ds(k * 16, 16)
                            blocks[p, t, pl.ds(D + k * 16, 16)] = \
                                sv * w[k] + b[k]
                            blocks[p, t, pl.ds(2 * D + k * 16, 16)] = \
                                rows_i[p, t, s]
                            blocks[p, t, pl.ds(3 * D + k * 16, 16)] = \
                                rows_t[p, t, s]
                    return carry2

                lax.fori_loop(0, C // 16, grp, 0)

                @pl.when(ch + 2 < nch)
                def _():
                    fire(ch + 2, p)

                pltpu.async_copy(blocks.at[p], out_hbm.at[pl.ds(base, C)],
                                 wsems[p])
            return carry

        lax.fori_loop(0, nch // 2, duo, 0)

        for p in range(2):
            ch = nch - 2 + p
            pltpu.make_async_copy(
                blocks.at[p], out_hbm.at[pl.ds(base0 + ch * C, C)],
                wsems[p]).wait()

    return body(sales_f, item2, text2, consts, tab_i, tab_t)


def kernel(sales, item_id, text, global_token, W_sales, b_sales,
           emb_item, emb_text):
    bsz, seq = item_id.shape
    n = bsz * seq
    nw = 32
    sales_f = (sales.reshape(bsz, seq).T
               .reshape(nw, n // (nw * C), C).astype(jnp.float32))
    item2 = item_id.T.reshape(nw, n // (nw * C), C).astype(jnp.int32)
    text2 = text.T.reshape(nw, n // (nw * C), C).astype(jnp.int32)
    consts = jnp.concatenate([
        global_token.reshape(-1).astype(jnp.float32),
        W_sales.reshape(-1).astype(jnp.float32),
        b_sales.reshape(-1).astype(jnp.float32),
    ])
    tab_i = _tc_repack(emb_item)
    tab_t = _tc_repack(emb_text)
    out = _sc_embed(sales_f, item2, text2, consts, tab_i, tab_t, n_tokens=n)
    return out.reshape(seq, bsz, 4 * D).transpose(1, 0, 2)
